# Initial kernel scaffold; baseline (speedup 1.0000x reference)
import jax
import jax.numpy as jnp
from jax.experimental import pallas as pl


def kernel(rawcount, W):
    raise NotImplementedError("write your pallas kernel here")


# speedup vs baseline: 3.2381x; 3.2381x over previous
import math

import jax
import jax.numpy as jnp
from jax import lax
from jax.experimental import pallas as pl
from jax.experimental.pallas import tpu as pltpu
from jax.experimental.pallas import tpu_sc as plsc

NUM_BUCKETS = 2048
MAX_VALUE = 20000
OUT_DIM = 64
BATCH = 4096
HIST = 200

NC = 2
NS = 16
NW = NC * NS
LANES = 16

B_TOTAL = BATCH * HIST
B_PER_W = B_TOTAL // NW
CHUNK = 1024
N_CHUNKS = B_PER_W // CHUNK
IDX_ROWS = CHUNK // 128


def _bucket_lut(anchor_i32):
    max_exact = NUM_BUCKETS // 2
    r = jnp.arange(MAX_VALUE, dtype=jnp.int32) + anchor_i32
    is_small = r < max_exact
    ratio = r.astype(jnp.float32) / float(max_exact)
    val_if_large = max_exact + (
        jnp.log(ratio) / math.log(MAX_VALUE / max_exact) * (NUM_BUCKETS - max_exact)
    ).astype(jnp.int32)
    val_if_large = jnp.minimum(val_if_large, NUM_BUCKETS - 1)
    return jnp.where(is_small, r, val_if_large)


def _sc_body(rc_hbm, w_hbm, lut_hbm, out_hbm, lut_v, rc_v, bkt_v, rows_v, sem):
    wid = lax.axis_index("s") * NC + lax.axis_index("c")
    base = wid * B_PER_W
    pltpu.sync_copy(lut_hbm, lut_v)

    def chunk_body(c, _):
        gbase = base + c * CHUNK
        pltpu.sync_copy(rc_hbm.at[pl.ds(gbase, CHUNK)], rc_v)

        def row_body(i, _):
            for j in range(128 // LANES):
                idx = rc_v[pl.ds(i * 128 + j * LANES, LANES)]
                bkt_v[i, pl.ds(j * LANES, LANES)] = plsc.load_gather(lut_v, [idx])
            return _

        lax.fori_loop(jnp.int32(0), jnp.int32(IDX_ROWS), row_body, None)

        handles = [
            pltpu.async_copy(
                w_hbm.at[bkt_v.at[jnp.int32(i)]],
                rows_v.at[pl.ds(jnp.int32(i * 128), 128)],
                sem,
            )
            for i in range(IDX_ROWS)
        ]
        for h in handles:
            h.wait()
        pltpu.sync_copy(rows_v, out_hbm.at[pl.ds(gbase, CHUNK)])
        return _

    lax.fori_loop(jnp.int32(0), jnp.int32(N_CHUNKS), chunk_body, None)


def kernel(rawcount, W):
    rc_flat = rawcount.astype(jnp.int32).reshape(B_TOTAL)
    lut = _bucket_lut(rc_flat[0] * 0)
    mesh = plsc.VectorSubcoreMesh(
        core_axis_name="c", subcore_axis_name="s", num_cores=NC, num_subcores=NS
    )
    out = pl.kernel(
        _sc_body,
        out_type=jax.ShapeDtypeStruct((B_TOTAL, OUT_DIM), jnp.float32),
        mesh=mesh,
        compiler_params=pltpu.CompilerParams(
            needs_layout_passes=False, use_tc_tiling_on_sc=False
        ),
        scratch_types=[
            pltpu.VMEM((MAX_VALUE,), jnp.int32),
            pltpu.VMEM((CHUNK,), jnp.int32),
            pltpu.VMEM((IDX_ROWS, 128), jnp.int32),
            pltpu.VMEM((CHUNK, OUT_DIM), jnp.float32),
            pltpu.SemaphoreType.DMA,
        ],
    )(rc_flat, W, lut)
    return out.reshape(BATCH, HIST, OUT_DIM)


# speedup vs baseline: 3.2736x; 1.0110x over previous
import math

import jax
import jax.numpy as jnp
from jax import lax
from jax.experimental import pallas as pl
from jax.experimental.pallas import tpu as pltpu
from jax.experimental.pallas import tpu_sc as plsc

NUM_BUCKETS = 2048
MAX_VALUE = 20000
OUT_DIM = 64
BATCH = 4096
HIST = 200

NC = 2
NS = 16
NW = NC * NS
LANES = 16

B_TOTAL = BATCH * HIST
B_PER_W = B_TOTAL // NW
CHUNK = 512
NBUF = 2
N_CHUNKS = B_PER_W // CHUNK
N_OUTER = N_CHUNKS // NBUF
IDX_ROWS = CHUNK // 128


def _bucket_lut(anchor_i32):
    max_exact = NUM_BUCKETS // 2
    r = jnp.arange(MAX_VALUE, dtype=jnp.int32) + anchor_i32
    is_small = r < max_exact
    ratio = r.astype(jnp.float32) / float(max_exact)
    val_if_large = max_exact + (
        jnp.log(ratio) / math.log(MAX_VALUE / max_exact) * (NUM_BUCKETS - max_exact)
    ).astype(jnp.int32)
    val_if_large = jnp.minimum(val_if_large, NUM_BUCKETS - 1)
    return jnp.where(is_small, r, val_if_large)


def _sc_body(
    rc_hbm, w_hbm, lut_hbm, out_hbm,
    lut_v, rc0, rc1, bkt0, bkt1, rows0, rows1, sem_g, so0, so1,
):
    rc_v = [rc0, rc1]
    bkt_v = [bkt0, bkt1]
    rows_v = [rows0, rows1]
    sem_out = [so0, so1]

    wid = lax.axis_index("s") * NC + lax.axis_index("c")
    base = wid * B_PER_W
    pltpu.sync_copy(lut_hbm, lut_v)

    def outer_body(c0, _):
        for b in range(NBUF):
            c = c0 * NBUF + b
            gbase = base + c * CHUNK

            @pl.when(c0 > jnp.int32(0))
            def _():
                pltpu.make_async_copy(
                    rows_v[b],
                    out_hbm.at[pl.ds(jnp.int32(0), CHUNK)],
                    sem_out[b],
                ).wait()

            pltpu.sync_copy(rc_hbm.at[pl.ds(gbase, CHUNK)], rc_v[b])

            def row_body(i, _, b=b):
                for j in range(128 // LANES):
                    idx = rc_v[b][pl.ds(i * 128 + j * LANES, LANES)]
                    bkt_v[b][i, pl.ds(j * LANES, LANES)] = plsc.load_gather(
                        lut_v, [idx]
                    )
                return _

            lax.fori_loop(jnp.int32(0), jnp.int32(IDX_ROWS), row_body, None)

            handles = [
                pltpu.async_copy(
                    w_hbm.at[bkt_v[b].at[jnp.int32(i)]],
                    rows_v[b].at[pl.ds(jnp.int32(i * 128), 128)],
                    sem_g,
                )
                for i in range(IDX_ROWS)
            ]
            for h in handles:
                h.wait()
            pltpu.async_copy(
                rows_v[b], out_hbm.at[pl.ds(gbase, CHUNK)], sem_out[b]
            )
        return _

    lax.fori_loop(jnp.int32(0), jnp.int32(N_OUTER), outer_body, None)
    for b in range(NBUF):
        pltpu.make_async_copy(
            rows_v[b], out_hbm.at[pl.ds(jnp.int32(0), CHUNK)], sem_out[b]
        ).wait()


def kernel(rawcount, W):
    rc_flat = rawcount.astype(jnp.int32).reshape(B_TOTAL)
    lut = _bucket_lut(rc_flat[0] * 0)
    mesh = plsc.VectorSubcoreMesh(
        core_axis_name="c", subcore_axis_name="s", num_cores=NC, num_subcores=NS
    )
    out = pl.kernel(
        _sc_body,
        out_type=jax.ShapeDtypeStruct((B_TOTAL, OUT_DIM), jnp.float32),
        mesh=mesh,
        compiler_params=pltpu.CompilerParams(
            needs_layout_passes=False, use_tc_tiling_on_sc=False
        ),
        scratch_types=[
            pltpu.VMEM((MAX_VALUE,), jnp.int32),
            pltpu.VMEM((CHUNK,), jnp.int32),
            pltpu.VMEM((CHUNK,), jnp.int32),
            pltpu.VMEM((IDX_ROWS, 128), jnp.int32),
            pltpu.VMEM((IDX_ROWS, 128), jnp.int32),
            pltpu.VMEM((CHUNK, OUT_DIM), jnp.float32),
            pltpu.VMEM((CHUNK, OUT_DIM), jnp.float32),
            pltpu.SemaphoreType.DMA,
            pltpu.SemaphoreType.DMA,
            pltpu.SemaphoreType.DMA,
        ],
    )(rc_flat, W, lut)
    return out.reshape(BATCH, HIST, OUT_DIM)


# speedup vs baseline: 4.7759x; 1.4589x over previous
import math

import jax
import jax.numpy as jnp
from jax import lax
from jax.experimental import pallas as pl
from jax.experimental.pallas import tpu as pltpu
from jax.experimental.pallas import tpu_sc as plsc

NUM_BUCKETS = 2048
MAX_VALUE = 20000
OUT_DIM = 64
BATCH = 4096
HIST = 200

NC = 2
NS = 16
NW = NC * NS
LANES = 16

ND = 8
NQ = NW // ND
H_PER_Q = HIST // NQ
DSUB = OUT_DIM // ND
NGRP = BATCH // LANES


def _bucket_lut(anchor_i32):
    max_exact = NUM_BUCKETS // 2
    r = jnp.arange(MAX_VALUE, dtype=jnp.int32) + anchor_i32
    is_small = r < max_exact
    ratio = r.astype(jnp.float32) / float(max_exact)
    val_if_large = max_exact + (
        jnp.log(ratio) / math.log(MAX_VALUE / max_exact) * (NUM_BUCKETS - max_exact)
    ).astype(jnp.int32)
    val_if_large = jnp.minimum(val_if_large, NUM_BUCKETS - 1)
    return jnp.where(is_small, r, val_if_large)


def _sc_body(
    rc_hbm, w_hbm, lut_hbm, out_hbm,
    lut_v, w_v, rc0, rc1, t0, t1, so0, so1,
):
    rc_v = [rc0, rc1]
    t_v = [t0, t1]
    sem_out = [so0, so1]

    wid = lax.axis_index("s") * NC + lax.axis_index("c")
    dt = lax.rem(wid, jnp.int32(ND))
    q = lax.div(wid, jnp.int32(ND))
    h0 = q * H_PER_Q
    d0 = dt * DSUB

    pltpu.sync_copy(lut_hbm, lut_v)
    pltpu.sync_copy(w_hbm.at[pl.ds(dt * (NUM_BUCKETS * DSUB), NUM_BUCKETS * DSUB)], w_v)

    def pair(c0, _):
        for bb in range(2):
            h = h0 + c0 * 2 + jnp.int32(bb)

            @pl.when(c0 > jnp.int32(0))
            def _(bb=bb):
                pltpu.make_async_copy(
                    t_v[bb],
                    out_hbm.at[pl.ds(jnp.int32(0), 1), pl.ds(jnp.int32(0), DSUB), :],
                    sem_out[bb],
                ).wait()

            pltpu.sync_copy(rc_hbm.at[pl.ds(h * BATCH, BATCH)], rc_v[bb])

            def grp(g, _, bb=bb):
                base = g * LANES
                rc16 = rc_v[bb][pl.ds(base, LANES)]
                bkt = plsc.load_gather(lut_v, [rc16])
                for ds in range(DSUB):
                    v = plsc.load_gather(w_v, [bkt + jnp.int32(ds * NUM_BUCKETS)])
                    t_v[bb][0, ds, pl.ds(base, LANES)] = v
                return _

            lax.fori_loop(jnp.int32(0), jnp.int32(NGRP), grp, None)
            pltpu.async_copy(
                t_v[bb],
                out_hbm.at[pl.ds(h, 1), pl.ds(d0, DSUB), :],
                sem_out[bb],
            )
        return _

    lax.fori_loop(jnp.int32(0), jnp.int32(H_PER_Q // 2), pair, None)
    for bb in range(2):
        pltpu.make_async_copy(
            t_v[bb],
            out_hbm.at[pl.ds(jnp.int32(0), 1), pl.ds(jnp.int32(0), DSUB), :],
            sem_out[bb],
        ).wait()


def kernel(rawcount, W):
    rc_lin = rawcount.astype(jnp.int32).T.reshape(HIST * BATCH)
    w_lin = jnp.transpose(W.reshape(NUM_BUCKETS, ND, DSUB), (1, 2, 0)).reshape(-1)
    lut = _bucket_lut(rc_lin[0] * 0)
    mesh = plsc.VectorSubcoreMesh(
        core_axis_name="c", subcore_axis_name="s", num_cores=NC, num_subcores=NS
    )
    out_p = pl.kernel(
        _sc_body,
        out_type=jax.ShapeDtypeStruct((HIST, OUT_DIM, BATCH), jnp.float32),
        mesh=mesh,
        compiler_params=pltpu.CompilerParams(
            needs_layout_passes=False, use_tc_tiling_on_sc=True
        ),
        scratch_types=[
            pltpu.VMEM((MAX_VALUE,), jnp.int32),
            pltpu.VMEM((NUM_BUCKETS * DSUB,), jnp.float32),
            pltpu.VMEM((BATCH,), jnp.int32),
            pltpu.VMEM((BATCH,), jnp.int32),
            pltpu.VMEM((1, DSUB, BATCH), jnp.float32),
            pltpu.VMEM((1, DSUB, BATCH), jnp.float32),
            pltpu.SemaphoreType.DMA,
            pltpu.SemaphoreType.DMA,
        ],
    )(rc_lin, w_lin, lut)
    return jnp.transpose(out_p, (2, 0, 1))


# speedup vs baseline: 16.3919x; 3.4322x over previous
import math

import jax
import jax.numpy as jnp
from jax import lax
from jax.experimental import pallas as pl
from jax.experimental.pallas import tpu as pltpu
from jax.experimental.pallas import tpu_sc as plsc

NUM_BUCKETS = 2048
MAX_VALUE = 20000
OUT_DIM = 64
BATCH = 4096
HIST = 200

NC = 2
NS = 16
NW = NC * NS
LANES = 16

ND = 8
NQ = NW // ND
H_PER_Q = HIST // NQ
DSUB = OUT_DIM // ND
NGRP = BATCH // LANES


def _bucket_lut(anchor_i32):
    max_exact = NUM_BUCKETS // 2
    r = jnp.arange(MAX_VALUE, dtype=jnp.int32) + anchor_i32
    is_small = r < max_exact
    ratio = r.astype(jnp.float32) / float(max_exact)
    val_if_large = max_exact + (
        jnp.log(ratio) / math.log(MAX_VALUE / max_exact) * (NUM_BUCKETS - max_exact)
    ).astype(jnp.int32)
    val_if_large = jnp.minimum(val_if_large, NUM_BUCKETS - 1)
    return jnp.where(is_small, r, val_if_large)


def _sc_body(
    rc_hbm, w_hbm, lut_hbm, out_hbm,
    lut_v, w_v, rc0, rc1, t0, t1, so0, so1,
):
    rc_v = [rc0, rc1]
    t_v = [t0, t1]
    sem_out = [so0, so1]

    wid = lax.axis_index("s") * NC + lax.axis_index("c")
    dt = lax.rem(wid, jnp.int32(ND))
    q = lax.div(wid, jnp.int32(ND))
    h0 = q * H_PER_Q
    d0 = dt * DSUB

    pltpu.sync_copy(lut_hbm, lut_v)
    pltpu.sync_copy(w_hbm.at[pl.ds(dt * (NUM_BUCKETS * DSUB), NUM_BUCKETS * DSUB)], w_v)

    def pair(c0, _):
        for bb in range(2):
            h = h0 + c0 * 2 + jnp.int32(bb)

            @pl.when(c0 > jnp.int32(0))
            def _(bb=bb):
                pltpu.make_async_copy(
                    t_v[bb],
                    out_hbm.at[pl.ds(jnp.int32(0), 1), pl.ds(jnp.int32(0), DSUB), :],
                    sem_out[bb],
                ).wait()

            pltpu.sync_copy(rc_hbm.at[pl.ds(h * BATCH, BATCH)], rc_v[bb])

            @plsc.parallel_loop(
                jnp.int32(0), jnp.int32(NGRP), step=jnp.int32(1), unroll=4
            )
            def grp(g, bb=bb):
                base = g * LANES
                rc16 = rc_v[bb][pl.ds(base, LANES)]
                bkt = plsc.load_gather(lut_v, [rc16])
                for ds in range(DSUB):
                    v = plsc.load_gather(w_v, [bkt + jnp.int32(ds * NUM_BUCKETS)])
                    t_v[bb][0, ds, pl.ds(base, LANES)] = v
            pltpu.async_copy(
                t_v[bb],
                out_hbm.at[pl.ds(h, 1), pl.ds(d0, DSUB), :],
                sem_out[bb],
            )
        return _

    lax.fori_loop(jnp.int32(0), jnp.int32(H_PER_Q // 2), pair, None)
    for bb in range(2):
        pltpu.make_async_copy(
            t_v[bb],
            out_hbm.at[pl.ds(jnp.int32(0), 1), pl.ds(jnp.int32(0), DSUB), :],
            sem_out[bb],
        ).wait()


def kernel(rawcount, W):
    rc_lin = rawcount.astype(jnp.int32).T.reshape(HIST * BATCH)
    w_lin = jnp.transpose(W.reshape(NUM_BUCKETS, ND, DSUB), (1, 2, 0)).reshape(-1)
    lut = _bucket_lut(rc_lin[0] * 0)
    mesh = plsc.VectorSubcoreMesh(
        core_axis_name="c", subcore_axis_name="s", num_cores=NC, num_subcores=NS
    )
    out_p = pl.kernel(
        _sc_body,
        out_type=jax.ShapeDtypeStruct((HIST, OUT_DIM, BATCH), jnp.float32),
        mesh=mesh,
        compiler_params=pltpu.CompilerParams(
            needs_layout_passes=False, use_tc_tiling_on_sc=True
        ),
        scratch_types=[
            pltpu.VMEM((MAX_VALUE,), jnp.int32),
            pltpu.VMEM((NUM_BUCKETS * DSUB,), jnp.float32),
            pltpu.VMEM((BATCH,), jnp.int32),
            pltpu.VMEM((BATCH,), jnp.int32),
            pltpu.VMEM((1, DSUB, BATCH), jnp.float32),
            pltpu.VMEM((1, DSUB, BATCH), jnp.float32),
            pltpu.SemaphoreType.DMA,
            pltpu.SemaphoreType.DMA,
        ],
    )(rc_lin, w_lin, lut)
    return jnp.transpose(out_p, (2, 0, 1))


# speedup vs baseline: 22.5238x; 1.3741x over previous
import math

import jax
import jax.numpy as jnp
from jax import lax
from jax.experimental import pallas as pl
from jax.experimental.pallas import tpu as pltpu
from jax.experimental.pallas import tpu_sc as plsc

NUM_BUCKETS = 2048
MAX_VALUE = 20000
OUT_DIM = 64
BATCH = 4096
HIST = 200

NC = 2
NS = 16
NW = NC * NS
LANES = 16

ND = 8
NQ = NW // ND
H_PER_Q = HIST // NQ
DSUB = OUT_DIM // ND
NGRP = BATCH // LANES


def _bucket_lut(anchor_i32):
    max_exact = NUM_BUCKETS // 2
    r = jnp.arange(MAX_VALUE, dtype=jnp.int32) + anchor_i32
    is_small = r < max_exact
    ratio = r.astype(jnp.float32) / float(max_exact)
    val_if_large = max_exact + (
        jnp.log(ratio) / math.log(MAX_VALUE / max_exact) * (NUM_BUCKETS - max_exact)
    ).astype(jnp.int32)
    val_if_large = jnp.minimum(val_if_large, NUM_BUCKETS - 1)
    return jnp.where(is_small, r, val_if_large)


def _sc_body(
    rc_hbm, w_hbm, lut_hbm, out_hbm,
    lut_v, w_v, rc0, rc1, t0, t1, so0, so1, sr0, sr1,
):
    rc_v = [rc0, rc1]
    t_v = [t0, t1]
    sem_out = [so0, so1]
    sem_rc = [sr0, sr1]

    wid = lax.axis_index("s") * NC + lax.axis_index("c")
    dt = lax.rem(wid, jnp.int32(ND))
    q = lax.div(wid, jnp.int32(ND))
    h0 = q * H_PER_Q
    d0 = dt * DSUB
    h_last = h0 + jnp.int32(H_PER_Q - 1)

    pltpu.sync_copy(lut_hbm, lut_v)
    pltpu.sync_copy(w_hbm.at[pl.ds(dt * (NUM_BUCKETS * DSUB), NUM_BUCKETS * DSUB)], w_v)
    pltpu.sync_copy(rc_hbm.at[pl.ds(h0 * BATCH, BATCH)], rc_v[0])

    def pair(c0, _):
        for bb in range(2):
            h = h0 + c0 * 2 + jnp.int32(bb)

            if bb == 0:
                @pl.when(c0 > jnp.int32(0))
                def _():
                    pltpu.make_async_copy(
                        rc_hbm.at[pl.ds(jnp.int32(0), BATCH)], rc_v[0], sem_rc[0]
                    ).wait()
            else:
                pltpu.make_async_copy(
                    rc_hbm.at[pl.ds(jnp.int32(0), BATCH)], rc_v[1], sem_rc[1]
                ).wait()

            hn = jnp.minimum(h + jnp.int32(1), h_last)
            pltpu.async_copy(
                rc_hbm.at[pl.ds(hn * BATCH, BATCH)], rc_v[bb ^ 1], sem_rc[bb ^ 1]
            )

            @pl.when(c0 > jnp.int32(0))
            def _(bb=bb):
                pltpu.make_async_copy(
                    t_v[bb],
                    out_hbm.at[pl.ds(jnp.int32(0), 1), pl.ds(jnp.int32(0), DSUB), :],
                    sem_out[bb],
                ).wait()

            @plsc.parallel_loop(
                jnp.int32(0), jnp.int32(NGRP), step=jnp.int32(1), unroll=8
            )
            def grp(g, bb=bb):
                base = g * LANES
                rc16 = rc_v[bb][pl.ds(base, LANES)]
                bkt = plsc.load_gather(lut_v, [rc16])
                for ds in range(DSUB):
                    v = plsc.load_gather(w_v, [bkt + jnp.int32(ds * NUM_BUCKETS)])
                    t_v[bb][0, ds, pl.ds(base, LANES)] = v
            pltpu.async_copy(
                t_v[bb],
                out_hbm.at[pl.ds(h, 1), pl.ds(d0, DSUB), :],
                sem_out[bb],
            )
        return _

    lax.fori_loop(jnp.int32(0), jnp.int32(H_PER_Q // 2), pair, None)
    pltpu.make_async_copy(
        rc_hbm.at[pl.ds(jnp.int32(0), BATCH)], rc_v[0], sem_rc[0]
    ).wait()
    for bb in range(2):
        pltpu.make_async_copy(
            t_v[bb],
            out_hbm.at[pl.ds(jnp.int32(0), 1), pl.ds(jnp.int32(0), DSUB), :],
            sem_out[bb],
        ).wait()


def kernel(rawcount, W):
    rc_lin = rawcount.astype(jnp.int32).T.reshape(HIST * BATCH)
    w_lin = jnp.transpose(W.reshape(NUM_BUCKETS, ND, DSUB), (1, 2, 0)).reshape(-1)
    lut = _bucket_lut(rc_lin[0] * 0)
    mesh = plsc.VectorSubcoreMesh(
        core_axis_name="c", subcore_axis_name="s", num_cores=NC, num_subcores=NS
    )
    out_p = pl.kernel(
        _sc_body,
        out_type=jax.ShapeDtypeStruct((HIST, OUT_DIM, BATCH), jnp.float32),
        mesh=mesh,
        compiler_params=pltpu.CompilerParams(
            needs_layout_passes=False, use_tc_tiling_on_sc=True
        ),
        scratch_types=[
            pltpu.VMEM((MAX_VALUE,), jnp.int32),
            pltpu.VMEM((NUM_BUCKETS * DSUB,), jnp.float32),
            pltpu.VMEM((BATCH,), jnp.int32),
            pltpu.VMEM((BATCH,), jnp.int32),
            pltpu.VMEM((1, DSUB, BATCH), jnp.float32),
            pltpu.VMEM((1, DSUB, BATCH), jnp.float32),
            pltpu.SemaphoreType.DMA,
            pltpu.SemaphoreType.DMA,
            pltpu.SemaphoreType.DMA,
            pltpu.SemaphoreType.DMA,
        ],
    )(rc_lin, w_lin, lut)
    return jnp.transpose(out_p, (2, 0, 1))
